# Initial kernel scaffold; baseline (speedup 1.0000x reference)
import jax
import jax.numpy as jnp
from jax.experimental import pallas as pl


def kernel(x, edge_index, W1, b1, W2, b2):
    raise NotImplementedError("write your pallas kernel here")


# speedup vs baseline: 14.1849x; 14.1849x over previous
import functools

import jax
import jax.numpy as jnp
from jax import lax
from jax.experimental import pallas as pl
from jax.experimental.pallas import tpu as pltpu
from jax.experimental.pallas import tpu_sc as plsc

N = 10000
NP = 10240
E = 320000
D = 128
D2 = 64
NC = 2
NS = 16
K = 80
NCH = (E // (NC * NS)) // K
NCH2 = (E // NS) // K
RPT = NP // NS
RB = 128
NRB = RPT // RB

_mesh = plsc.VectorSubcoreMesh(
    core_axis_name="c", subcore_axis_name="s", num_cores=NC, num_subcores=NS
)


def _sc_deg(dst_r, zeros_np, ones_k):
    @functools.partial(
        pl.kernel,
        out_type=jax.ShapeDtypeStruct((NC, NP), jnp.float32),
        mesh=_mesh,
        scratch_types=[
            pltpu.VMEM((NCH, K), jnp.int32),
            pltpu.VMEM((K,), jnp.float32),
            pltpu.VMEM((RPT,), jnp.float32),
            pltpu.VMEM_SHARED((NP,), jnp.float32),
        ],
    )
    def k(dst_hbm, zeros_hbm, ones_hbm, out_hbm, idx_v, ones_v, rbuf, acc):
        c = lax.axis_index("c")
        s = lax.axis_index("s")
        base = s * RPT
        pltpu.sync_copy(zeros_hbm.at[pl.ds(base, RPT)], rbuf)
        pltpu.sync_copy(rbuf, acc.at[pl.ds(base, RPT)])
        pltpu.sync_copy(ones_hbm, ones_v)
        pltpu.sync_copy(dst_hbm.at[c, s], idx_v)
        plsc.subcore_barrier()

        def step(j, carry):
            pltpu.sync_copy(ones_v, acc.at[idx_v.at[j]], add=True)
            return carry

        lax.fori_loop(0, NCH, step, 0)
        plsc.subcore_barrier()
        pltpu.sync_copy(acc.at[pl.ds(base, RPT)], rbuf)
        pltpu.sync_copy(rbuf, out_hbm.at[c, pl.ds(base, RPT)])

    return k(dst_r, zeros_np, ones_k)


def _sc_agg(ysplit, srco, dst_a):
    @functools.partial(
        pl.kernel,
        out_type=jax.ShapeDtypeStruct((NC, NP, D2), jnp.float32),
        mesh=_mesh,
        scratch_types=[
            pltpu.VMEM((NCH2, K), jnp.int32),
            pltpu.VMEM((NCH2, K), jnp.int32),
            pltpu.VMEM((K, D2), jnp.float32),
            pltpu.VMEM((RB, D2), jnp.float32),
            pltpu.VMEM_SHARED((NP, D2), jnp.float32),
            pltpu.SemaphoreType.DMA,
        ],
        compiler_params=pltpu.CompilerParams(use_tc_tiling_on_sc=False),
    )
    def k(y_hbm, src_hbm, dst_hbm, out_hbm, sidx, didx, gbuf, cbuf, acc, sem):
        c = lax.axis_index("c")
        s = lax.axis_index("s")
        base = s * RPT
        for t in range(NRB):
            pltpu.sync_copy(y_hbm.at[pl.ds(c * NP + base + t * RB, RB)], cbuf)
            pltpu.sync_copy(cbuf, acc.at[pl.ds(base + t * RB, RB)])
        pltpu.sync_copy(src_hbm.at[c, s], sidx)
        pltpu.sync_copy(dst_hbm.at[s], didx)
        plsc.subcore_barrier()

        def step(j, carry):
            pltpu.async_copy(y_hbm.at[sidx.at[j]], gbuf, sem).wait()
            pltpu.sync_copy(gbuf, acc.at[didx.at[j]], add=True)
            return carry

        lax.fori_loop(0, NCH2, step, 0)
        plsc.subcore_barrier()
        for t in range(NRB):
            pltpu.sync_copy(acc.at[pl.ds(base + t * RB, RB)], cbuf)
            pltpu.sync_copy(cbuf, out_hbm.at[c, pl.ds(base + t * RB, RB)])

    return k(ysplit, srco, dst_a)


R = 512
GRID = NP // R


def _tc_scale_matmul(xp, W1, degp):
    def body(x_ref, w_ref, p_ref, o_ref):
        p = p_ref[...]
        z = lax.rsqrt(p[0] + p[1] + 1.0)
        y = (
            jnp.dot(x_ref[...], w_ref[...], preferred_element_type=jnp.float32)
            * z[:, None]
        )
        o_ref[...] = jnp.stack([y[:, :D2], y[:, D2:]], axis=0)

    return pl.pallas_call(
        body,
        grid=(GRID,),
        in_specs=[
            pl.BlockSpec((R, D), lambda i: (i, 0)),
            pl.BlockSpec((D, D), lambda i: (0, 0)),
            pl.BlockSpec((NC, R), lambda i: (0, i)),
        ],
        out_specs=pl.BlockSpec((NC, R, D2), lambda i: (0, i, 0)),
        out_shape=jax.ShapeDtypeStruct((NC, NP, D2), jnp.float32),
    )(xp, W1, degp)


def _tc_mid(parts, degp, b1r, W2):
    def body(p_ref, dp_ref, b_ref, w_ref, o_ref):
        dp = dp_ref[...]
        z = lax.rsqrt(dp[0] + dp[1] + 1.0)
        p = p_ref[...]
        agg = jnp.concatenate([p[0], p[1]], axis=1)
        h = jnp.maximum(agg * z[:, None] + b_ref[...], 0.0)
        y = jnp.dot(h, w_ref[...], preferred_element_type=jnp.float32) * z[:, None]
        o_ref[...] = jnp.stack([y[:, :D2], y[:, D2:]], axis=0)

    return pl.pallas_call(
        body,
        grid=(GRID,),
        in_specs=[
            pl.BlockSpec((NC, R, D2), lambda i: (0, i, 0)),
            pl.BlockSpec((NC, R), lambda i: (0, i)),
            pl.BlockSpec((1, D), lambda i: (0, 0)),
            pl.BlockSpec((D, D), lambda i: (0, 0)),
        ],
        out_specs=pl.BlockSpec((NC, R, D2), lambda i: (0, i, 0)),
        out_shape=jax.ShapeDtypeStruct((NC, NP, D2), jnp.float32),
    )(parts, degp, b1r, W2)


def _tc_final(parts, degp, b2r):
    def body(p_ref, dp_ref, b_ref, o_ref):
        dp = dp_ref[...]
        z = lax.rsqrt(dp[0] + dp[1] + 1.0)
        p = p_ref[...]
        agg = jnp.concatenate([p[0], p[1]], axis=1)
        o_ref[...] = agg * z[:, None] + b_ref[...]

    return pl.pallas_call(
        body,
        grid=(GRID,),
        in_specs=[
            pl.BlockSpec((NC, R, D2), lambda i: (0, i, 0)),
            pl.BlockSpec((NC, R), lambda i: (0, i)),
            pl.BlockSpec((1, D), lambda i: (0, 0)),
        ],
        out_specs=pl.BlockSpec((R, D), lambda i: (i, 0)),
        out_shape=jax.ShapeDtypeStruct((NP, D), jnp.float32),
    )(parts, degp, b2r)


def kernel(x, edge_index, W1, b1, W2, b2):
    ei = edge_index.astype(jnp.int32)
    src_t = ei[0].reshape(NS, NCH2, K)
    srco = jnp.stack([src_t, src_t + NP], axis=0)
    dst_a = ei[1].reshape(NS, NCH2, K)
    dst_deg = ei[1].reshape(NC, NS, NCH, K)
    xp = jnp.pad(x, ((0, NP - N), (0, 0)))
    zeros_np = jnp.zeros((NP,), jnp.float32)
    ones_k = jnp.ones((K,), jnp.float32)

    degp = _sc_deg(dst_deg, zeros_np, ones_k)
    y1 = _tc_scale_matmul(xp, W1, degp).reshape(NC * NP, D2)
    p1 = _sc_agg(y1, srco, dst_a)
    y2 = _tc_mid(p1, degp, b1.reshape(1, D), W2).reshape(NC * NP, D2)
    p2 = _sc_agg(y2, srco, dst_a)
    out = _tc_final(p2, degp, b2.reshape(1, D))
    return out[:N]


# speedup vs baseline: 27.3513x; 1.9282x over previous
import functools

import jax
import jax.numpy as jnp
from jax import lax
from jax.experimental import pallas as pl
from jax.experimental.pallas import tpu as pltpu
from jax.experimental.pallas import tpu_sc as plsc

N = 10000
NP = 10240
E = 320000
D = 128
D2 = 64
NC = 2
NS = 16
K = 80
NCH = (E // (NC * NS)) // K
NCH2 = (E // NS) // K
RPT = NP // NS
RB = 128
NRB = RPT // RB
NB = 5

_mesh = plsc.VectorSubcoreMesh(
    core_axis_name="c", subcore_axis_name="s", num_cores=NC, num_subcores=NS
)


def _sc_deg(dst_r, zeros_np, ones_k):
    @functools.partial(
        pl.kernel,
        out_type=jax.ShapeDtypeStruct((NC, NP), jnp.float32),
        mesh=_mesh,
        scratch_types=[
            pltpu.VMEM((NCH, K), jnp.int32),
            pltpu.VMEM((K,), jnp.float32),
            pltpu.VMEM((RPT,), jnp.float32),
            pltpu.VMEM_SHARED((NP,), jnp.float32),
        ],
    )
    def k(dst_hbm, zeros_hbm, ones_hbm, out_hbm, idx_v, ones_v, rbuf, acc):
        c = lax.axis_index("c")
        s = lax.axis_index("s")
        base = s * RPT
        pltpu.sync_copy(zeros_hbm.at[pl.ds(base, RPT)], rbuf)
        pltpu.sync_copy(rbuf, acc.at[pl.ds(base, RPT)])
        pltpu.sync_copy(ones_hbm, ones_v)
        pltpu.sync_copy(dst_hbm.at[c, s], idx_v)
        plsc.subcore_barrier()

        def step(j, carry):
            pltpu.sync_copy(ones_v, acc.at[idx_v.at[j]], add=True)
            return carry

        lax.fori_loop(0, NCH, step, 0)
        plsc.subcore_barrier()
        pltpu.sync_copy(acc.at[pl.ds(base, RPT)], rbuf)
        pltpu.sync_copy(rbuf, out_hbm.at[c, pl.ds(base, RPT)])

    return k(dst_r, zeros_np, ones_k)


def _sc_agg(ysplit, srco, dst_a):
    @functools.partial(
        pl.kernel,
        out_type=jax.ShapeDtypeStruct((NC, NP, D2), jnp.float32),
        mesh=_mesh,
        scratch_types=[
            pltpu.VMEM((NCH2, K), jnp.int32),
            pltpu.VMEM((NCH2, K), jnp.int32),
            [pltpu.VMEM((K, D2), jnp.float32) for _ in range(NB)],
            pltpu.VMEM((RB, D2), jnp.float32),
            pltpu.VMEM_SHARED((NP, D2), jnp.float32),
            [pltpu.SemaphoreType.DMA for _ in range(NB)],
            [pltpu.SemaphoreType.DMA for _ in range(NB)],
        ],
        compiler_params=pltpu.CompilerParams(use_tc_tiling_on_sc=False),
    )
    def k(y_hbm, src_hbm, dst_hbm, out_hbm, sidx, didx, gbufs, cbuf, acc,
          gsems, tsems):
        c = lax.axis_index("c")
        s = lax.axis_index("s")
        base = s * RPT
        for t in range(NRB):
            pltpu.sync_copy(y_hbm.at[pl.ds(c * NP + base + t * RB, RB)], cbuf)
            pltpu.sync_copy(cbuf, acc.at[pl.ds(base + t * RB, RB)])
        pltpu.sync_copy(src_hbm.at[c, s], sidx)
        pltpu.sync_copy(dst_hbm.at[s], didx)
        plsc.subcore_barrier()

        for b in range(NB):
            pltpu.async_copy(y_hbm.at[sidx.at[b]], gbufs[b], gsems[b])

        def group(jo, last):
            scats = []
            for b in range(NB):
                pltpu.make_async_copy(
                    y_hbm.at[sidx.at[jo + b]], gbufs[b], gsems[b]
                ).wait()
                scats.append(pltpu.async_copy(
                    gbufs[b], acc.at[didx.at[jo + b]], tsems[b], add=True
                ))
            for b in range(NB):
                scats[b].wait()
                if not last:
                    pltpu.async_copy(
                        y_hbm.at[sidx.at[jo + NB + b]], gbufs[b], gsems[b]
                    )

        def outer(o, carry):
            group(o * NB, last=False)
            return carry

        lax.fori_loop(0, NCH2 // NB - 1, outer, 0)
        group(NCH2 - NB, last=True)
        plsc.subcore_barrier()
        for t in range(NRB):
            pltpu.sync_copy(acc.at[pl.ds(base + t * RB, RB)], cbuf)
            pltpu.sync_copy(cbuf, out_hbm.at[c, pl.ds(base + t * RB, RB)])

    return k(ysplit, srco, dst_a)


R = 512
GRID = NP // R


def _tc_scale_matmul(xp, W1, degp):
    def body(x_ref, w_ref, p_ref, o_ref):
        p = p_ref[...]
        z = lax.rsqrt(p[0] + p[1] + 1.0)
        y = (
            jnp.dot(x_ref[...], w_ref[...], preferred_element_type=jnp.float32)
            * z[:, None]
        )
        o_ref[...] = jnp.stack([y[:, :D2], y[:, D2:]], axis=0)

    return pl.pallas_call(
        body,
        grid=(GRID,),
        in_specs=[
            pl.BlockSpec((R, D), lambda i: (i, 0)),
            pl.BlockSpec((D, D), lambda i: (0, 0)),
            pl.BlockSpec((NC, R), lambda i: (0, i)),
        ],
        out_specs=pl.BlockSpec((NC, R, D2), lambda i: (0, i, 0)),
        out_shape=jax.ShapeDtypeStruct((NC, NP, D2), jnp.float32),
    )(xp, W1, degp)


def _tc_mid(parts, degp, b1r, W2):
    def body(p_ref, dp_ref, b_ref, w_ref, o_ref):
        dp = dp_ref[...]
        z = lax.rsqrt(dp[0] + dp[1] + 1.0)
        p = p_ref[...]
        agg = jnp.concatenate([p[0], p[1]], axis=1)
        h = jnp.maximum(agg * z[:, None] + b_ref[...], 0.0)
        y = jnp.dot(h, w_ref[...], preferred_element_type=jnp.float32) * z[:, None]
        o_ref[...] = jnp.stack([y[:, :D2], y[:, D2:]], axis=0)

    return pl.pallas_call(
        body,
        grid=(GRID,),
        in_specs=[
            pl.BlockSpec((NC, R, D2), lambda i: (0, i, 0)),
            pl.BlockSpec((NC, R), lambda i: (0, i)),
            pl.BlockSpec((1, D), lambda i: (0, 0)),
            pl.BlockSpec((D, D), lambda i: (0, 0)),
        ],
        out_specs=pl.BlockSpec((NC, R, D2), lambda i: (0, i, 0)),
        out_shape=jax.ShapeDtypeStruct((NC, NP, D2), jnp.float32),
    )(parts, degp, b1r, W2)


def _tc_final(parts, degp, b2r):
    def body(p_ref, dp_ref, b_ref, o_ref):
        dp = dp_ref[...]
        z = lax.rsqrt(dp[0] + dp[1] + 1.0)
        p = p_ref[...]
        agg = jnp.concatenate([p[0], p[1]], axis=1)
        o_ref[...] = agg * z[:, None] + b_ref[...]

    return pl.pallas_call(
        body,
        grid=(GRID,),
        in_specs=[
            pl.BlockSpec((NC, R, D2), lambda i: (0, i, 0)),
            pl.BlockSpec((NC, R), lambda i: (0, i)),
            pl.BlockSpec((1, D), lambda i: (0, 0)),
        ],
        out_specs=pl.BlockSpec((R, D), lambda i: (i, 0)),
        out_shape=jax.ShapeDtypeStruct((NP, D), jnp.float32),
    )(parts, degp, b2r)


def kernel(x, edge_index, W1, b1, W2, b2):
    ei = edge_index.astype(jnp.int32)
    src_t = ei[0].reshape(NS, NCH2, K)
    srco = jnp.stack([src_t, src_t + NP], axis=0)
    dst_a = ei[1].reshape(NS, NCH2, K)
    dst_deg = ei[1].reshape(NC, NS, NCH, K)
    xp = jnp.pad(x, ((0, NP - N), (0, 0)))
    zeros_np = jnp.zeros((NP,), jnp.float32)
    ones_k = jnp.ones((K,), jnp.float32)

    degp = _sc_deg(dst_deg, zeros_np, ones_k)
    y1 = _tc_scale_matmul(xp, W1, degp).reshape(NC * NP, D2)
    p1 = _sc_agg(y1, srco, dst_a)
    y2 = _tc_mid(p1, degp, b1.reshape(1, D), W2).reshape(NC * NP, D2)
    p2 = _sc_agg(y2, srco, dst_a)
    out = _tc_final(p2, degp, b2.reshape(1, D))
    return out[:N]


# speedup vs baseline: 27.5344x; 1.0067x over previous
import functools

import jax
import jax.numpy as jnp
from jax import lax
from jax.experimental import pallas as pl
from jax.experimental.pallas import tpu as pltpu
from jax.experimental.pallas import tpu_sc as plsc

N = 10000
NP = 10240
E = 320000
D = 128
D2 = 64
NC = 2
NS = 16
K = 80
NCH = (E // (NC * NS)) // K
NCH2 = (E // NS) // K
RPT = NP // NS
RPA = N // NS
NB = 5
NBD = 5

_mesh = plsc.VectorSubcoreMesh(
    core_axis_name="c", subcore_axis_name="s", num_cores=NC, num_subcores=NS
)


def _sc_deg(dst_r, zeros_np, ones_k):
    @functools.partial(
        pl.kernel,
        out_type=jax.ShapeDtypeStruct((NC, NP), jnp.float32),
        mesh=_mesh,
        scratch_types=[
            pltpu.VMEM((NCH, K), jnp.int32),
            pltpu.VMEM((K,), jnp.float32),
            pltpu.VMEM_SHARED((NP,), jnp.float32),
            [pltpu.SemaphoreType.DMA for _ in range(NBD)],
        ],
    )
    def k(dst_hbm, zeros_hbm, ones_hbm, out_hbm, idx_v, ones_v, acc, dsems):
        c = lax.axis_index("c")
        s = lax.axis_index("s")
        base = s * RPT
        pltpu.sync_copy(zeros_hbm.at[pl.ds(base, RPT)], acc.at[pl.ds(base, RPT)])
        pltpu.sync_copy(ones_hbm, ones_v)
        pltpu.sync_copy(dst_hbm.at[c, s], idx_v)
        plsc.subcore_barrier()

        def group(o, carry):
            jo = o * NBD
            descs = [
                pltpu.async_copy(
                    ones_v, acc.at[idx_v.at[jo + b]], dsems[b], add=True
                )
                for b in range(NBD)
            ]
            for d in descs:
                d.wait()
            return carry

        lax.fori_loop(0, NCH // NBD, group, 0)
        plsc.subcore_barrier()
        pltpu.sync_copy(acc.at[pl.ds(base, RPT)], out_hbm.at[c, pl.ds(base, RPT)])

    return k(dst_r, zeros_np, ones_k)


def _sc_agg(ysplit, srco, dst_a):
    @functools.partial(
        pl.kernel,
        out_type=jax.ShapeDtypeStruct((NC, N, D2), jnp.float32),
        mesh=_mesh,
        scratch_types=[
            pltpu.VMEM((NCH2 * K,), jnp.int32),
            pltpu.VMEM((NCH2, K), jnp.int32),
            [pltpu.VMEM((K, D2), jnp.float32) for _ in range(NB)],
            pltpu.VMEM_SHARED((N, D2), jnp.float32),
            [pltpu.SemaphoreType.DMA for _ in range(NB)],
            [pltpu.SemaphoreType.DMA for _ in range(NB)],
        ],
        compiler_params=pltpu.CompilerParams(use_tc_tiling_on_sc=False),
    )
    def k(y_hbm, src_hbm, dst_hbm, out_hbm, sidx, didx, gbufs, acc,
          gsems, tsems):
        c = lax.axis_index("c")
        s = lax.axis_index("s")
        base = s * RPA
        pltpu.sync_copy(
            y_hbm.at[pl.ds(c * N + base, RPA)], acc.at[pl.ds(base, RPA)]
        )
        pltpu.sync_copy(src_hbm.at[c, s], sidx)
        pltpu.sync_copy(dst_hbm.at[s], didx)
        plsc.subcore_barrier()

        for b in range(NB):
            pltpu.async_copy(y_hbm.at[sidx.at[pl.ds(b * K, K)]], gbufs[b], gsems[b])

        def group(jo, last):
            scats = []
            for b in range(NB):
                pltpu.make_async_copy(
                    y_hbm.at[sidx.at[pl.ds((jo + b) * K, K)]], gbufs[b], gsems[b]
                ).wait()
                scats.append(pltpu.async_copy(
                    gbufs[b], acc.at[didx.at[jo + b]], tsems[b], add=True
                ))
            for b in range(NB):
                scats[b].wait()
                if not last:
                    pltpu.async_copy(
                        y_hbm.at[sidx.at[pl.ds((jo + NB + b) * K, K)]],
                        gbufs[b], gsems[b]
                    )

        def outer(o, carry):
            group(o * NB, last=False)
            return carry

        lax.fori_loop(0, NCH2 // NB - 1, outer, 0)
        group(NCH2 - NB, last=True)
        plsc.subcore_barrier()
        pltpu.sync_copy(
            acc.at[pl.ds(base, RPA)], out_hbm.at[c, pl.ds(base, RPA)]
        )

    return k(ysplit, srco, dst_a)


R = 400
GRID = N // R


def _tc_scale_matmul(x, W1, degp):
    def body(x_ref, w_ref, p_ref, o_ref):
        p = p_ref[...]
        z = lax.rsqrt(p[:, 0] + p[:, 1] + 1.0)
        y = (
            jnp.dot(x_ref[...], w_ref[...], preferred_element_type=jnp.float32)
            * z[:, None]
        )
        o_ref[...] = jnp.stack([y[:, :D2], y[:, D2:]], axis=0)

    return pl.pallas_call(
        body,
        grid=(GRID,),
        in_specs=[
            pl.BlockSpec((R, D), lambda i: (i, 0)),
            pl.BlockSpec((D, D), lambda i: (0, 0)),
            pl.BlockSpec((R, NC), lambda i: (i, 0)),
        ],
        out_specs=pl.BlockSpec((NC, R, D2), lambda i: (0, i, 0)),
        out_shape=jax.ShapeDtypeStruct((NC, N, D2), jnp.float32),
    )(x, W1, degp)


def _tc_mid(parts, degp, b1r, W2):
    def body(p_ref, dp_ref, b_ref, w_ref, o_ref):
        dp = dp_ref[...]
        z = lax.rsqrt(dp[:, 0] + dp[:, 1] + 1.0)
        p = p_ref[...]
        agg = jnp.concatenate([p[0], p[1]], axis=1)
        h = jnp.maximum(agg * z[:, None] + b_ref[...], 0.0)
        y = jnp.dot(h, w_ref[...], preferred_element_type=jnp.float32) * z[:, None]
        o_ref[...] = jnp.stack([y[:, :D2], y[:, D2:]], axis=0)

    return pl.pallas_call(
        body,
        grid=(GRID,),
        in_specs=[
            pl.BlockSpec((NC, R, D2), lambda i: (0, i, 0)),
            pl.BlockSpec((R, NC), lambda i: (i, 0)),
            pl.BlockSpec((1, D), lambda i: (0, 0)),
            pl.BlockSpec((D, D), lambda i: (0, 0)),
        ],
        out_specs=pl.BlockSpec((NC, R, D2), lambda i: (0, i, 0)),
        out_shape=jax.ShapeDtypeStruct((NC, N, D2), jnp.float32),
    )(parts, degp, b1r, W2)


def _tc_final(parts, degp, b2r):
    def body(p_ref, dp_ref, b_ref, o_ref):
        dp = dp_ref[...]
        z = lax.rsqrt(dp[:, 0] + dp[:, 1] + 1.0)
        p = p_ref[...]
        agg = jnp.concatenate([p[0], p[1]], axis=1)
        o_ref[...] = agg * z[:, None] + b_ref[...]

    return pl.pallas_call(
        body,
        grid=(GRID,),
        in_specs=[
            pl.BlockSpec((NC, R, D2), lambda i: (0, i, 0)),
            pl.BlockSpec((R, NC), lambda i: (i, 0)),
            pl.BlockSpec((1, D), lambda i: (0, 0)),
        ],
        out_specs=pl.BlockSpec((R, D), lambda i: (i, 0)),
        out_shape=jax.ShapeDtypeStruct((N, D), jnp.float32),
    )(parts, degp, b2r)


def kernel(x, edge_index, W1, b1, W2, b2):
    ei = edge_index.astype(jnp.int32)
    src_t = ei[0].reshape(NS, NCH2 * K)
    srco = jnp.stack([src_t, src_t + N], axis=0)
    dst_a = ei[1].reshape(NS, NCH2, K)
    dst_deg = ei[1].reshape(NC, NS, NCH, K)
    zeros_np = jnp.zeros((NP,), jnp.float32)
    ones_k = jnp.ones((K,), jnp.float32)

    degp = _sc_deg(dst_deg, zeros_np, ones_k).T
    y1 = _tc_scale_matmul(x, W1, degp).reshape(NC * N, D2)
    p1 = _sc_agg(y1, srco, dst_a)
    y2 = _tc_mid(p1, degp, b1.reshape(1, D), W2).reshape(NC * N, D2)
    p2 = _sc_agg(y2, srco, dst_a)
    return _tc_final(p2, degp, b2.reshape(1, D))


# speedup vs baseline: 30.7680x; 1.1174x over previous
import functools

import jax
import jax.numpy as jnp
from jax import lax
from jax.experimental import pallas as pl
from jax.experimental.pallas import tpu as pltpu
from jax.experimental.pallas import tpu_sc as plsc

N = 10000
NP = 10240
E = 320000
D = 128
D2 = 64
NC = 2
NS = 16
K = 80
NCH = (E // (NC * NS)) // K
NCH2 = (E // NS) // K
RPT = NP // NS
RPA = N // NS
NB = 5
NWB = 5
WBK = RPA // NWB
NBD = 5

_mesh = plsc.VectorSubcoreMesh(
    core_axis_name="c", subcore_axis_name="s", num_cores=NC, num_subcores=NS
)


def _sc_deg(dst_r, zeros_np, ones_k):
    @functools.partial(
        pl.kernel,
        out_type=jax.ShapeDtypeStruct((NC, NP), jnp.float32),
        mesh=_mesh,
        scratch_types=[
            pltpu.VMEM((NCH, K), jnp.int32),
            pltpu.VMEM((K,), jnp.float32),
            pltpu.VMEM_SHARED((NP,), jnp.float32),
            [pltpu.SemaphoreType.DMA for _ in range(NBD)],
        ],
    )
    def k(dst_hbm, zeros_hbm, ones_hbm, out_hbm, idx_v, ones_v, acc, dsems):
        c = lax.axis_index("c")
        s = lax.axis_index("s")
        base = s * RPT
        pltpu.sync_copy(zeros_hbm.at[pl.ds(base, RPT)], acc.at[pl.ds(base, RPT)])
        pltpu.sync_copy(ones_hbm, ones_v)
        pltpu.sync_copy(dst_hbm.at[c, s], idx_v)
        plsc.subcore_barrier()

        def group(o, carry):
            jo = o * NBD
            descs = [
                pltpu.async_copy(
                    ones_v, acc.at[idx_v.at[jo + b]], dsems[b], add=True
                )
                for b in range(NBD)
            ]
            for d in descs:
                d.wait()
            return carry

        lax.fori_loop(0, NCH // NBD, group, 0)
        plsc.subcore_barrier()
        pltpu.sync_copy(acc.at[pl.ds(base, RPT)], out_hbm.at[c, pl.ds(base, RPT)])

    return k(dst_r, zeros_np, ones_k)


def _sc_agg(yv, srco, dst_a, zeros_nd, wbidx):
    @functools.partial(
        pl.kernel,
        out_type=jax.ShapeDtypeStruct((2 * N, D2), jnp.float32),
        mesh=_mesh,
        scratch_types=[
            pltpu.VMEM((NCH2 * K,), jnp.int32),
            pltpu.VMEM((NCH2, K), jnp.int32),
            pltpu.VMEM((NWB, WBK), jnp.int32),
            pltpu.VMEM((WBK, D2), jnp.float32),
            [pltpu.VMEM((K, D2), jnp.float32) for _ in range(NB)],
            pltpu.VMEM_SHARED((N, D2), jnp.float32),
            [pltpu.SemaphoreType.DMA for _ in range(NB)],
            [pltpu.SemaphoreType.DMA for _ in range(NB)],
        ],
        compiler_params=pltpu.CompilerParams(use_tc_tiling_on_sc=False),
    )
    def k(y_hbm, src_hbm, dst_hbm, zeros_hbm, wb_hbm, out_hbm,
          sidx, didx, wbi, wbuf, gbufs, acc, gsems, tsems):
        c = lax.axis_index("c")
        s = lax.axis_index("s")
        base = s * RPA
        pltpu.sync_copy(zeros_hbm.at[pl.ds(base, RPA)], acc.at[pl.ds(base, RPA)])
        pltpu.sync_copy(src_hbm.at[c, s], sidx)
        pltpu.sync_copy(dst_hbm.at[s], didx)
        pltpu.sync_copy(wb_hbm.at[c, s], wbi)
        plsc.subcore_barrier()

        for b in range(NB):
            pltpu.async_copy(y_hbm.at[sidx.at[pl.ds(b * K, K)]], gbufs[b], gsems[b])

        def group(jo, last):
            scats = []
            for b in range(NB):
                pltpu.make_async_copy(
                    y_hbm.at[sidx.at[pl.ds((jo + b) * K, K)]], gbufs[b], gsems[b]
                ).wait()
                scats.append(pltpu.async_copy(
                    gbufs[b], acc.at[didx.at[jo + b]], tsems[b], add=True
                ))
            for b in range(NB):
                scats[b].wait()
                if not last:
                    pltpu.async_copy(
                        y_hbm.at[sidx.at[pl.ds((jo + NB + b) * K, K)]],
                        gbufs[b], gsems[b]
                    )

        def outer(o, carry):
            group(o * NB, last=False)
            return carry

        lax.fori_loop(0, NCH2 // NB - 1, outer, 0)
        group(NCH2 - NB, last=True)
        plsc.subcore_barrier()
        for tch in range(NWB):
            pltpu.sync_copy(acc.at[pl.ds(base + tch * WBK, WBK)], wbuf)
            pltpu.sync_copy(wbuf, out_hbm.at[wbi.at[tch]])

    return k(yv, srco, dst_a, zeros_nd, wbidx)


R = 400
GRID = N // R


def _tc_scale_matmul(x, W1, degp):
    def body(x_ref, w_ref, p_ref, o_ref):
        p = p_ref[...]
        z = lax.rsqrt(p[:, 0] + p[:, 1] + 1.0)
        o_ref[...] = (
            jnp.dot(x_ref[...], w_ref[...], preferred_element_type=jnp.float32)
            * z[:, None]
        )

    return pl.pallas_call(
        body,
        grid=(GRID,),
        in_specs=[
            pl.BlockSpec((R, D), lambda i: (i, 0)),
            pl.BlockSpec((D, D), lambda i: (0, 0)),
            pl.BlockSpec((R, NC), lambda i: (i, 0)),
        ],
        out_specs=pl.BlockSpec((R, D), lambda i: (i, 0)),
        out_shape=jax.ShapeDtypeStruct((N, D), jnp.float32),
    )(x, W1, degp)


def _tc_mid(p1, y1, degp, b1r, W2):
    def body(p_ref, y_ref, dp_ref, b_ref, w_ref, o_ref):
        dp = dp_ref[...]
        z = lax.rsqrt(dp[:, 0] + dp[:, 1] + 1.0)
        agg = p_ref[...] + y_ref[...]
        h = jnp.maximum(agg * z[:, None] + b_ref[...], 0.0)
        o_ref[...] = (
            jnp.dot(h, w_ref[...], preferred_element_type=jnp.float32) * z[:, None]
        )

    return pl.pallas_call(
        body,
        grid=(GRID,),
        in_specs=[
            pl.BlockSpec((R, D), lambda i: (i, 0)),
            pl.BlockSpec((R, D), lambda i: (i, 0)),
            pl.BlockSpec((R, NC), lambda i: (i, 0)),
            pl.BlockSpec((1, D), lambda i: (0, 0)),
            pl.BlockSpec((D, D), lambda i: (0, 0)),
        ],
        out_specs=pl.BlockSpec((R, D), lambda i: (i, 0)),
        out_shape=jax.ShapeDtypeStruct((N, D), jnp.float32),
    )(p1, y1, degp, b1r, W2)


def _tc_final(p2, y2, degp, b2r):
    def body(p_ref, y_ref, dp_ref, b_ref, o_ref):
        dp = dp_ref[...]
        z = lax.rsqrt(dp[:, 0] + dp[:, 1] + 1.0)
        o_ref[...] = (p_ref[...] + y_ref[...]) * z[:, None] + b_ref[...]

    return pl.pallas_call(
        body,
        grid=(GRID,),
        in_specs=[
            pl.BlockSpec((R, D), lambda i: (i, 0)),
            pl.BlockSpec((R, D), lambda i: (i, 0)),
            pl.BlockSpec((R, NC), lambda i: (i, 0)),
            pl.BlockSpec((1, D), lambda i: (0, 0)),
        ],
        out_specs=pl.BlockSpec((R, D), lambda i: (i, 0)),
        out_shape=jax.ShapeDtypeStruct((N, D), jnp.float32),
    )(p2, y2, degp, b2r)


def kernel(x, edge_index, W1, b1, W2, b2):
    ei = edge_index.astype(jnp.int32)
    src_t = ei[0].reshape(NS, NCH2 * K)
    srco = jnp.stack([2 * src_t, 2 * src_t + 1], axis=0)
    dst_a = ei[1].reshape(NS, NCH2, K)
    dst_deg = ei[1].reshape(NC, NS, NCH, K)
    zeros_np = jnp.zeros((NP,), jnp.float32)
    zeros_nd = jnp.zeros((N, D2), jnp.float32)
    ones_k = jnp.ones((K,), jnp.float32)
    iota = jnp.arange(N, dtype=jnp.int32).reshape(NS, NWB, WBK)
    wbidx = jnp.stack([2 * iota, 2 * iota + 1], axis=0)

    degp = _sc_deg(dst_deg, zeros_np, ones_k).T
    y1 = _tc_scale_matmul(x, W1, degp)
    p1 = _sc_agg(y1.reshape(2 * N, D2), srco, dst_a, zeros_nd, wbidx)
    y2 = _tc_mid(p1.reshape(N, D), y1, degp, b1.reshape(1, D), W2)
    p2 = _sc_agg(y2.reshape(2 * N, D2), srco, dst_a, zeros_nd, wbidx)
    return _tc_final(p2.reshape(N, D), y2, degp, b2.reshape(1, D))


# speedup vs baseline: 32.1459x; 1.0448x over previous
import functools

import jax
import jax.numpy as jnp
from jax import lax
from jax.experimental import pallas as pl
from jax.experimental.pallas import tpu as pltpu
from jax.experimental.pallas import tpu_sc as plsc

N = 10000
NP = 10240
E = 320000
D = 128
D2 = 64
NC = 2
NS = 16
K = 80
NCH = (E // (NC * NS)) // K
EPT = E // NS
KA = 128
NCH2 = EPT // KA
KT = EPT - NCH2 * KA
RPT = NP // NS
RPA = N // NS
NB = 4
NWB = 5
WBK = RPA // NWB
NBD = 5

_mesh = plsc.VectorSubcoreMesh(
    core_axis_name="c", subcore_axis_name="s", num_cores=NC, num_subcores=NS
)


def _sc_deg(dst_r, zeros_np, ones_k):
    @functools.partial(
        pl.kernel,
        out_type=jax.ShapeDtypeStruct((NC, NP), jnp.float32),
        mesh=_mesh,
        scratch_types=[
            pltpu.VMEM((NCH, K), jnp.int32),
            pltpu.VMEM((K,), jnp.float32),
            pltpu.VMEM_SHARED((NP,), jnp.float32),
            [pltpu.SemaphoreType.DMA for _ in range(NBD)],
        ],
    )
    def k(dst_hbm, zeros_hbm, ones_hbm, out_hbm, idx_v, ones_v, acc, dsems):
        c = lax.axis_index("c")
        s = lax.axis_index("s")
        base = s * RPT
        pltpu.sync_copy(zeros_hbm.at[pl.ds(base, RPT)], acc.at[pl.ds(base, RPT)])
        pltpu.sync_copy(ones_hbm, ones_v)
        pltpu.sync_copy(dst_hbm.at[c, s], idx_v)
        plsc.subcore_barrier()

        def group(o, carry):
            jo = o * NBD
            descs = [
                pltpu.async_copy(
                    ones_v, acc.at[idx_v.at[jo + b]], dsems[b], add=True
                )
                for b in range(NBD)
            ]
            for d in descs:
                d.wait()
            return carry

        lax.fori_loop(0, NCH // NBD, group, 0)
        plsc.subcore_barrier()
        pltpu.sync_copy(acc.at[pl.ds(base, RPT)], out_hbm.at[c, pl.ds(base, RPT)])

    return k(dst_r, zeros_np, ones_k)


def _sc_agg(yv, srco, srct, dst_a, dst_t, zeros_nd, wbidx):
    @functools.partial(
        pl.kernel,
        out_type=jax.ShapeDtypeStruct((2 * N, D2), jnp.float32),
        mesh=_mesh,
        scratch_types=[
            pltpu.VMEM((NCH2 * KA,), jnp.int32),
            pltpu.VMEM((KT,), jnp.int32),
            pltpu.VMEM((NCH2, KA), jnp.int32),
            pltpu.VMEM((1, KT), jnp.int32),
            pltpu.VMEM((NWB, WBK), jnp.int32),
            pltpu.VMEM((WBK, D2), jnp.float32),
            [pltpu.VMEM((KA, D2), jnp.float32) for _ in range(NB)],
            pltpu.VMEM_SHARED((N, D2), jnp.float32),
            [pltpu.SemaphoreType.DMA for _ in range(NB)],
            [pltpu.SemaphoreType.DMA for _ in range(NB)],
        ],
        compiler_params=pltpu.CompilerParams(use_tc_tiling_on_sc=False),
    )
    def k(y_hbm, src_hbm, srct_hbm, dst_hbm, dstt_hbm, zeros_hbm, wb_hbm,
          out_hbm, sidx, sidxt, didx, didxt, wbi, wbuf, gbufs, acc,
          gsems, tsems):
        c = lax.axis_index("c")
        s = lax.axis_index("s")
        base = s * RPA
        pltpu.sync_copy(zeros_hbm.at[pl.ds(base, RPA)], acc.at[pl.ds(base, RPA)])
        pltpu.sync_copy(src_hbm.at[c, s], sidx)
        pltpu.sync_copy(srct_hbm.at[c, s], sidxt)
        pltpu.sync_copy(dst_hbm.at[s], didx)
        pltpu.sync_copy(dstt_hbm.at[s], didxt)
        pltpu.sync_copy(wb_hbm.at[c, s], wbi)
        plsc.subcore_barrier()

        for b in range(NB):
            pltpu.async_copy(
                y_hbm.at[sidx.at[pl.ds(b * KA, KA)]], gbufs[b], gsems[b]
            )

        def group(jo, last):
            scats = []
            for b in range(NB):
                pltpu.make_async_copy(
                    y_hbm.at[sidx.at[pl.ds((jo + b) * KA, KA)]], gbufs[b],
                    gsems[b]
                ).wait()
                scats.append(pltpu.async_copy(
                    gbufs[b], acc.at[didx.at[jo + b]], tsems[b], add=True
                ))
            for b in range(NB):
                scats[b].wait()
                if not last:
                    pltpu.async_copy(
                        y_hbm.at[sidx.at[pl.ds((jo + NB + b) * KA, KA)]],
                        gbufs[b], gsems[b]
                    )

        def outer(o, carry):
            group(o * NB, last=False)
            return carry

        lax.fori_loop(0, NCH2 // NB - 1, outer, 0)
        group(NCH2 - NB, last=True)
        pltpu.async_copy(
            y_hbm.at[sidxt], gbufs[0].at[pl.ds(0, KT)], gsems[0]
        ).wait()
        pltpu.async_copy(
            gbufs[0].at[pl.ds(0, KT)], acc.at[didxt.at[0]], tsems[0], add=True
        ).wait()
        plsc.subcore_barrier()
        for tch in range(NWB):
            pltpu.sync_copy(acc.at[pl.ds(base + tch * WBK, WBK)], wbuf)
            pltpu.sync_copy(wbuf, out_hbm.at[wbi.at[tch]])

    return k(yv, srco, srct, dst_a, dst_t, zeros_nd, wbidx)


R = 1000
GRID = N // R


def _tc_matmul(x, W1):
    def body(x_ref, w_ref, o_ref):
        o_ref[...] = jnp.dot(
            x_ref[...], w_ref[...], preferred_element_type=jnp.float32
        )

    return pl.pallas_call(
        body,
        grid=(GRID,),
        in_specs=[
            pl.BlockSpec((R, D), lambda i: (i, 0)),
            pl.BlockSpec((D, D), lambda i: (0, 0)),
        ],
        out_specs=pl.BlockSpec((R, D), lambda i: (i, 0)),
        out_shape=jax.ShapeDtypeStruct((N, D), jnp.float32),
    )(x, W1)


def _tc_scale(xw, degp):
    def body(x_ref, p_ref, o_ref):
        p = p_ref[...]
        z = lax.rsqrt(p[:, 0] + p[:, 1] + 1.0)
        o_ref[...] = x_ref[...] * z[:, None]

    return pl.pallas_call(
        body,
        grid=(GRID,),
        in_specs=[
            pl.BlockSpec((R, D), lambda i: (i, 0)),
            pl.BlockSpec((R, NC), lambda i: (i, 0)),
        ],
        out_specs=pl.BlockSpec((R, D), lambda i: (i, 0)),
        out_shape=jax.ShapeDtypeStruct((N, D), jnp.float32),
    )(xw, degp)


def _tc_mid(p1, y1, degp, b1r, W2):
    def body(p_ref, y_ref, dp_ref, b_ref, w_ref, o_ref):
        dp = dp_ref[...]
        z = lax.rsqrt(dp[:, 0] + dp[:, 1] + 1.0)
        agg = p_ref[...] + y_ref[...]
        h = jnp.maximum(agg * z[:, None] + b_ref[...], 0.0)
        o_ref[...] = (
            jnp.dot(h, w_ref[...], preferred_element_type=jnp.float32) * z[:, None]
        )

    return pl.pallas_call(
        body,
        grid=(GRID,),
        in_specs=[
            pl.BlockSpec((R, D), lambda i: (i, 0)),
            pl.BlockSpec((R, D), lambda i: (i, 0)),
            pl.BlockSpec((R, NC), lambda i: (i, 0)),
            pl.BlockSpec((1, D), lambda i: (0, 0)),
            pl.BlockSpec((D, D), lambda i: (0, 0)),
        ],
        out_specs=pl.BlockSpec((R, D), lambda i: (i, 0)),
        out_shape=jax.ShapeDtypeStruct((N, D), jnp.float32),
    )(p1, y1, degp, b1r, W2)


def _tc_final(p2, y2, degp, b2r):
    def body(p_ref, y_ref, dp_ref, b_ref, o_ref):
        dp = dp_ref[...]
        z = lax.rsqrt(dp[:, 0] + dp[:, 1] + 1.0)
        o_ref[...] = (p_ref[...] + y_ref[...]) * z[:, None] + b_ref[...]

    return pl.pallas_call(
        body,
        grid=(GRID,),
        in_specs=[
            pl.BlockSpec((R, D), lambda i: (i, 0)),
            pl.BlockSpec((R, D), lambda i: (i, 0)),
            pl.BlockSpec((R, NC), lambda i: (i, 0)),
            pl.BlockSpec((1, D), lambda i: (0, 0)),
        ],
        out_specs=pl.BlockSpec((R, D), lambda i: (i, 0)),
        out_shape=jax.ShapeDtypeStruct((N, D), jnp.float32),
    )(p2, y2, degp, b2r)


def kernel(x, edge_index, W1, b1, W2, b2):
    ei = edge_index.astype(jnp.int32)
    src_t = ei[0].reshape(NS, EPT)
    src2 = 2 * src_t
    srco = jnp.stack([src2[:, :NCH2 * KA], src2[:, :NCH2 * KA] + 1], axis=0)
    srct = jnp.stack([src2[:, NCH2 * KA:], src2[:, NCH2 * KA:] + 1], axis=0)
    dst_r = ei[1].reshape(NS, EPT)
    dst_a = dst_r[:, :NCH2 * KA].reshape(NS, NCH2, KA)
    dst_t = dst_r[:, NCH2 * KA:].reshape(NS, 1, KT)
    dst_deg = ei[1].reshape(NC, NS, NCH, K)
    zeros_np = jnp.zeros((NP,), jnp.float32)
    zeros_nd = jnp.zeros((N, D2), jnp.float32)
    ones_k = jnp.ones((K,), jnp.float32)
    iota = jnp.arange(N, dtype=jnp.int32).reshape(NS, NWB, WBK)
    wbidx = jnp.stack([2 * iota, 2 * iota + 1], axis=0)

    xw1 = _tc_matmul(x, W1)
    degp = _sc_deg(dst_deg, zeros_np, ones_k).T
    y1 = _tc_scale(xw1, degp)
    p1 = _sc_agg(y1.reshape(2 * N, D2), srco, srct, dst_a, dst_t,
                 zeros_nd, wbidx)
    y2 = _tc_mid(p1.reshape(N, D), y1, degp, b1.reshape(1, D), W2)
    p2 = _sc_agg(y2.reshape(2 * N, D2), srco, srct, dst_a, dst_t,
                 zeros_nd, wbidx)
    return _tc_final(p2.reshape(N, D), y2, degp, b2.reshape(1, D))


# speedup vs baseline: 32.9163x; 1.0240x over previous
import functools

import jax
import jax.numpy as jnp
from jax import lax
from jax.experimental import pallas as pl
from jax.experimental.pallas import tpu as pltpu
from jax.experimental.pallas import tpu_sc as plsc

N = 10000
NP = 10240
E = 320000
D = 128
D2 = 64
NC = 2
NS = 16
K = 80
NCH = (E // (NC * NS)) // K
EPT = E // NS
KA = 128
NCH2 = EPT // KA
KT = EPT - NCH2 * KA
RPT = NP // NS
RPA = N // NS
NB = 4
NWB = 5
WBK = RPA // NWB
NBD = 5

_mesh = plsc.VectorSubcoreMesh(
    core_axis_name="c", subcore_axis_name="s", num_cores=NC, num_subcores=NS
)


def _sc_deg(dst_r, zeros_np, ones_k):
    @functools.partial(
        pl.kernel,
        out_type=jax.ShapeDtypeStruct((NC, NP), jnp.float32),
        mesh=_mesh,
        scratch_types=[
            pltpu.VMEM((NCH, K), jnp.int32),
            pltpu.VMEM((K,), jnp.float32),
            pltpu.VMEM_SHARED((NP,), jnp.float32),
            [pltpu.SemaphoreType.DMA for _ in range(NBD)],
        ],
    )
    def k(dst_hbm, zeros_hbm, ones_hbm, out_hbm, idx_v, ones_v, acc, dsems):
        c = lax.axis_index("c")
        s = lax.axis_index("s")
        base = s * RPT
        pltpu.sync_copy(zeros_hbm.at[pl.ds(base, RPT)], acc.at[pl.ds(base, RPT)])
        pltpu.sync_copy(ones_hbm, ones_v)
        pltpu.sync_copy(dst_hbm.at[c, s], idx_v)
        plsc.subcore_barrier()

        def group(o, carry):
            jo = o * NBD
            descs = [
                pltpu.async_copy(
                    ones_v, acc.at[idx_v.at[jo + b]], dsems[b], add=True
                )
                for b in range(NBD)
            ]
            for d in descs:
                d.wait()
            return carry

        lax.fori_loop(0, NCH // NBD, group, 0)
        plsc.subcore_barrier()
        pltpu.sync_copy(acc.at[pl.ds(base, RPT)], out_hbm.at[c, pl.ds(base, RPT)])

    return k(dst_r, zeros_np, ones_k)


def _sc_agg(yv, srco, srct, dst_a, dst_t, zeros_nd, wbidx):
    @functools.partial(
        pl.kernel,
        out_type=jax.ShapeDtypeStruct((2 * N, D2), jnp.float32),
        mesh=_mesh,
        scratch_types=[
            pltpu.VMEM((NCH2 * KA,), jnp.int32),
            pltpu.VMEM((KT,), jnp.int32),
            pltpu.VMEM((NCH2, KA), jnp.int32),
            pltpu.VMEM((1, KT), jnp.int32),
            pltpu.VMEM((NWB, WBK), jnp.int32),
            pltpu.VMEM((WBK, D2), jnp.float32),
            [pltpu.VMEM((KA, D2), jnp.float32) for _ in range(NB)],
            pltpu.VMEM_SHARED((N, D2), jnp.float32),
            [pltpu.SemaphoreType.DMA for _ in range(NB)],
            [pltpu.SemaphoreType.DMA for _ in range(NB)],
        ],
        compiler_params=pltpu.CompilerParams(use_tc_tiling_on_sc=False),
    )
    def k(y_hbm, src_hbm, srct_hbm, dst_hbm, dstt_hbm, zeros_hbm, wb_hbm,
          out_hbm, sidx, sidxt, didx, didxt, wbi, wbuf, gbufs, acc,
          gsems, tsems):
        c = lax.axis_index("c")
        s = lax.axis_index("s")
        base = s * RPA
        pltpu.sync_copy(zeros_hbm.at[pl.ds(base, RPA)], acc.at[pl.ds(base, RPA)])
        pltpu.sync_copy(src_hbm.at[c, s], sidx)
        pltpu.sync_copy(srct_hbm.at[c, s], sidxt)
        pltpu.sync_copy(dst_hbm.at[s], didx)
        pltpu.sync_copy(dstt_hbm.at[s], didxt)
        pltpu.sync_copy(wb_hbm.at[c, s], wbi)
        plsc.subcore_barrier()

        for b in range(NB):
            pltpu.async_copy(
                y_hbm.at[sidx.at[pl.ds(b * KA, KA)]], gbufs[b], gsems[b]
            )

        def group(jo, last):
            scats = []
            for b in range(NB):
                pltpu.make_async_copy(
                    y_hbm.at[sidx.at[pl.ds((jo + b) * KA, KA)]], gbufs[b],
                    gsems[b]
                ).wait()
                scats.append(pltpu.async_copy(
                    gbufs[b], acc.at[didx.at[jo + b]], tsems[b], add=True
                ))
            for b in range(NB):
                scats[b].wait()
                if not last:
                    pltpu.async_copy(
                        y_hbm.at[sidx.at[pl.ds((jo + NB + b) * KA, KA)]],
                        gbufs[b], gsems[b]
                    )

        def outer(o, carry):
            group(o * NB, last=False)
            return carry

        lax.fori_loop(0, NCH2 // NB - 1, outer, 0)
        group(NCH2 - NB, last=True)
        pltpu.async_copy(
            y_hbm.at[sidxt], gbufs[0].at[pl.ds(0, KT)], gsems[0]
        ).wait()
        pltpu.async_copy(
            gbufs[0].at[pl.ds(0, KT)], acc.at[didxt.at[0]], tsems[0], add=True
        ).wait()
        plsc.subcore_barrier()
        for tch in range(NWB):
            pltpu.sync_copy(acc.at[pl.ds(base + tch * WBK, WBK)], wbuf)
            pltpu.sync_copy(wbuf, out_hbm.at[wbi.at[tch]])

    return k(yv, srco, srct, dst_a, dst_t, zeros_nd, wbidx)


R = 1000
GRID = N // R


def _tc_scale_matmul(x, W1, degp):
    def body(x_ref, w_ref, p_ref, o_ref):
        p = p_ref[...]
        z = lax.rsqrt(p[:, 0] + p[:, 1] + 1.0)
        o_ref[...] = (
            jnp.dot(x_ref[...], w_ref[...], preferred_element_type=jnp.float32)
            * z[:, None]
        )

    return pl.pallas_call(
        body,
        grid=(GRID,),
        in_specs=[
            pl.BlockSpec((R, D), lambda i: (i, 0)),
            pl.BlockSpec((D, D), lambda i: (0, 0)),
            pl.BlockSpec((R, NC), lambda i: (i, 0)),
        ],
        out_specs=pl.BlockSpec((R, D), lambda i: (i, 0)),
        out_shape=jax.ShapeDtypeStruct((N, D), jnp.float32),
    )(x, W1, degp)


def _tc_mid(p1, y1, degp, b1r, W2):
    def body(p_ref, y_ref, dp_ref, b_ref, w_ref, o_ref):
        dp = dp_ref[...]
        z = lax.rsqrt(dp[:, 0] + dp[:, 1] + 1.0)
        agg = p_ref[...] + y_ref[...]
        h = jnp.maximum(agg * z[:, None] + b_ref[...], 0.0)
        o_ref[...] = (
            jnp.dot(h, w_ref[...], preferred_element_type=jnp.float32) * z[:, None]
        )

    return pl.pallas_call(
        body,
        grid=(GRID,),
        in_specs=[
            pl.BlockSpec((R, D), lambda i: (i, 0)),
            pl.BlockSpec((R, D), lambda i: (i, 0)),
            pl.BlockSpec((R, NC), lambda i: (i, 0)),
            pl.BlockSpec((1, D), lambda i: (0, 0)),
            pl.BlockSpec((D, D), lambda i: (0, 0)),
        ],
        out_specs=pl.BlockSpec((R, D), lambda i: (i, 0)),
        out_shape=jax.ShapeDtypeStruct((N, D), jnp.float32),
    )(p1, y1, degp, b1r, W2)


def _tc_final(p2, y2, degp, b2r):
    def body(p_ref, y_ref, dp_ref, b_ref, o_ref):
        dp = dp_ref[...]
        z = lax.rsqrt(dp[:, 0] + dp[:, 1] + 1.0)
        o_ref[...] = (p_ref[...] + y_ref[...]) * z[:, None] + b_ref[...]

    return pl.pallas_call(
        body,
        grid=(GRID,),
        in_specs=[
            pl.BlockSpec((R, D), lambda i: (i, 0)),
            pl.BlockSpec((R, D), lambda i: (i, 0)),
            pl.BlockSpec((R, NC), lambda i: (i, 0)),
            pl.BlockSpec((1, D), lambda i: (0, 0)),
        ],
        out_specs=pl.BlockSpec((R, D), lambda i: (i, 0)),
        out_shape=jax.ShapeDtypeStruct((N, D), jnp.float32),
    )(p2, y2, degp, b2r)


def kernel(x, edge_index, W1, b1, W2, b2):
    ei = edge_index.astype(jnp.int32)
    src_t = ei[0].reshape(NS, EPT)
    src2 = 2 * src_t
    srco = jnp.stack([src2[:, :NCH2 * KA], src2[:, :NCH2 * KA] + 1], axis=0)
    srct = jnp.stack([src2[:, NCH2 * KA:], src2[:, NCH2 * KA:] + 1], axis=0)
    dst_r = ei[1].reshape(NS, EPT)
    dst_a = dst_r[:, :NCH2 * KA].reshape(NS, NCH2, KA)
    dst_t = dst_r[:, NCH2 * KA:].reshape(NS, 1, KT)
    dst_deg = ei[1].reshape(NC, NS, NCH, K)
    zeros_np = jnp.zeros((NP,), jnp.float32)
    zeros_nd = jnp.zeros((N, D2), jnp.float32)
    ones_k = jnp.ones((K,), jnp.float32)
    iota = jnp.arange(N, dtype=jnp.int32).reshape(NS, NWB, WBK)
    wbidx = jnp.stack([2 * iota, 2 * iota + 1], axis=0)
    srco, srct, wbidx = lax.optimization_barrier((srco, srct, wbidx))

    degp = _sc_deg(dst_deg, zeros_np, ones_k).T
    y1 = _tc_scale_matmul(x, W1, degp)
    p1 = _sc_agg(y1.reshape(2 * N, D2), srco, srct, dst_a, dst_t,
                 zeros_nd, wbidx)
    y2 = _tc_mid(p1.reshape(N, D), y1, degp, b1.reshape(1, D), W2)
    p2 = _sc_agg(y2.reshape(2 * N, D2), srco, srct, dst_a, dst_t,
                 zeros_nd, wbidx)
    return _tc_final(p2.reshape(N, D), y2, degp, b2.reshape(1, D))


# speedup vs baseline: 33.8499x; 1.0284x over previous
import functools

import jax
import jax.numpy as jnp
from jax import lax
from jax.experimental import pallas as pl
from jax.experimental.pallas import tpu as pltpu
from jax.experimental.pallas import tpu_sc as plsc

N = 10000
NP = 10240
E = 320000
D = 128
D2 = 64
NC = 2
NS = 16
K = 80
NCH = (E // (NC * NS)) // K
EPT = E // NS
KA = 128
NCH2 = EPT // KA
KT = EPT - NCH2 * KA
RPT = NP // NS
RPA = N // NS
NB = 4
NWB = 5
WBK = RPA // NWB
NBD = 5

_mesh = plsc.VectorSubcoreMesh(
    core_axis_name="c", subcore_axis_name="s", num_cores=NC, num_subcores=NS
)


def _sc_deg(dst_r, zeros_np, ones_k):
    @functools.partial(
        pl.kernel,
        out_type=jax.ShapeDtypeStruct((NC, NP), jnp.float32),
        mesh=_mesh,
        scratch_types=[
            pltpu.VMEM((NCH, K), jnp.int32),
            pltpu.VMEM((K,), jnp.float32),
            pltpu.VMEM_SHARED((NP,), jnp.float32),
            [pltpu.SemaphoreType.DMA for _ in range(NBD)],
        ],
    )
    def k(dst_hbm, zeros_hbm, ones_hbm, out_hbm, idx_v, ones_v, acc, dsems):
        c = lax.axis_index("c")
        s = lax.axis_index("s")
        base = s * RPT
        pltpu.sync_copy(zeros_hbm.at[pl.ds(base, RPT)], acc.at[pl.ds(base, RPT)])
        pltpu.sync_copy(ones_hbm, ones_v)
        pltpu.sync_copy(dst_hbm.at[c, s], idx_v)
        plsc.subcore_barrier()

        def group(o, carry):
            jo = o * NBD
            descs = [
                pltpu.async_copy(
                    ones_v, acc.at[idx_v.at[jo + b]], dsems[b], add=True
                )
                for b in range(NBD)
            ]
            for d in descs:
                d.wait()
            return carry

        lax.fori_loop(0, NCH // NBD, group, 0)
        plsc.subcore_barrier()
        pltpu.sync_copy(acc.at[pl.ds(base, RPT)], out_hbm.at[c, pl.ds(base, RPT)])

    return k(dst_r, zeros_np, ones_k)


def _sc_agg(yv, srco, srct, dst_a, dst_t, zeros_nd, wbidx):
    @functools.partial(
        pl.kernel,
        out_type=jax.ShapeDtypeStruct((2 * N, D2), jnp.float32),
        mesh=_mesh,
        scratch_types=[
            pltpu.VMEM((NCH2 * KA,), jnp.int32),
            pltpu.VMEM((KT,), jnp.int32),
            pltpu.VMEM((NCH2, KA), jnp.int32),
            pltpu.VMEM((1, KT), jnp.int32),
            pltpu.VMEM((NWB, WBK), jnp.int32),
            [pltpu.VMEM((WBK, D2), jnp.float32) for _ in range(2)],
            [pltpu.VMEM((KA, D2), jnp.float32) for _ in range(NB)],
            pltpu.VMEM_SHARED((N, D2), jnp.float32),
            [pltpu.SemaphoreType.DMA for _ in range(NB)],
            [pltpu.SemaphoreType.DMA for _ in range(NB)],
        ],
        compiler_params=pltpu.CompilerParams(use_tc_tiling_on_sc=False),
    )
    def k(y_hbm, src_hbm, srct_hbm, dst_hbm, dstt_hbm, zeros_hbm, wb_hbm,
          out_hbm, sidx, sidxt, didx, didxt, wbi, wbufs, gbufs, acc,
          gsems, tsems):
        c = lax.axis_index("c")
        s = lax.axis_index("s")
        base = s * RPA
        inits = [
            pltpu.async_copy(zeros_hbm.at[pl.ds(base, RPA)],
                             acc.at[pl.ds(base, RPA)], gsems[0]),
            pltpu.async_copy(src_hbm.at[c, s], sidx, gsems[1]),
            pltpu.async_copy(srct_hbm.at[c, s], sidxt, gsems[2]),
            pltpu.async_copy(dst_hbm.at[s], didx, gsems[3]),
            pltpu.async_copy(dstt_hbm.at[s], didxt, tsems[0]),
            pltpu.async_copy(wb_hbm.at[c, s], wbi, tsems[1]),
        ]
        for d in inits:
            d.wait()
        plsc.subcore_barrier()

        for b in range(NB):
            pltpu.async_copy(
                y_hbm.at[sidx.at[pl.ds(b * KA, KA)]], gbufs[b], gsems[b]
            )

        def group(jo, last):
            scats = []
            for b in range(NB):
                pltpu.make_async_copy(
                    y_hbm.at[sidx.at[pl.ds((jo + b) * KA, KA)]], gbufs[b],
                    gsems[b]
                ).wait()
                scats.append(pltpu.async_copy(
                    gbufs[b], acc.at[didx.at[jo + b]], tsems[b], add=True
                ))
            for b in range(NB):
                scats[b].wait()
                if not last:
                    pltpu.async_copy(
                        y_hbm.at[sidx.at[pl.ds((jo + NB + b) * KA, KA)]],
                        gbufs[b], gsems[b]
                    )

        def outer(o, carry):
            group(o * NB, last=False)
            return carry

        lax.fori_loop(0, NCH2 // NB - 1, outer, 0)
        group(NCH2 - NB, last=True)
        pltpu.async_copy(
            y_hbm.at[sidxt], gbufs[0].at[pl.ds(0, KT)], gsems[0]
        ).wait()
        pltpu.async_copy(
            gbufs[0].at[pl.ds(0, KT)], acc.at[didxt.at[0]], tsems[0], add=True
        ).wait()
        plsc.subcore_barrier()
        scouts = [None, None]
        for tch in range(NWB):
            w = wbufs[tch % 2]
            if scouts[tch % 2] is not None:
                scouts[tch % 2].wait()
            pltpu.sync_copy(acc.at[pl.ds(base + tch * WBK, WBK)], w)
            scouts[tch % 2] = pltpu.async_copy(
                w, out_hbm.at[wbi.at[tch]], tsems[tch % 2]
            )
        for d in scouts:
            d.wait()

    return k(yv, srco, srct, dst_a, dst_t, zeros_nd, wbidx)


R = 1000
GRID = N // R


def _tc_scale_matmul(x, W1, degp):
    def body(x_ref, w_ref, p_ref, o_ref):
        p = p_ref[...]
        z = lax.rsqrt(p[:, 0] + p[:, 1] + 1.0)
        o_ref[...] = (
            jnp.dot(x_ref[...], w_ref[...], preferred_element_type=jnp.float32)
            * z[:, None]
        )

    return pl.pallas_call(
        body,
        grid=(GRID,),
        in_specs=[
            pl.BlockSpec((R, D), lambda i: (i, 0)),
            pl.BlockSpec((D, D), lambda i: (0, 0)),
            pl.BlockSpec((R, NC), lambda i: (i, 0)),
        ],
        out_specs=pl.BlockSpec((R, D), lambda i: (i, 0)),
        out_shape=jax.ShapeDtypeStruct((N, D), jnp.float32),
    )(x, W1, degp)


def _tc_mid(p1, y1, degp, b1r, W2):
    def body(p_ref, y_ref, dp_ref, b_ref, w_ref, o_ref):
        dp = dp_ref[...]
        z = lax.rsqrt(dp[:, 0] + dp[:, 1] + 1.0)
        agg = p_ref[...] + y_ref[...]
        h = jnp.maximum(agg * z[:, None] + b_ref[...], 0.0)
        o_ref[...] = (
            jnp.dot(h, w_ref[...], preferred_element_type=jnp.float32) * z[:, None]
        )

    return pl.pallas_call(
        body,
        grid=(GRID,),
        in_specs=[
            pl.BlockSpec((R, D), lambda i: (i, 0)),
            pl.BlockSpec((R, D), lambda i: (i, 0)),
            pl.BlockSpec((R, NC), lambda i: (i, 0)),
            pl.BlockSpec((1, D), lambda i: (0, 0)),
            pl.BlockSpec((D, D), lambda i: (0, 0)),
        ],
        out_specs=pl.BlockSpec((R, D), lambda i: (i, 0)),
        out_shape=jax.ShapeDtypeStruct((N, D), jnp.float32),
    )(p1, y1, degp, b1r, W2)


def _tc_final(p2, y2, degp, b2r):
    def body(p_ref, y_ref, dp_ref, b_ref, o_ref):
        dp = dp_ref[...]
        z = lax.rsqrt(dp[:, 0] + dp[:, 1] + 1.0)
        o_ref[...] = (p_ref[...] + y_ref[...]) * z[:, None] + b_ref[...]

    return pl.pallas_call(
        body,
        grid=(GRID,),
        in_specs=[
            pl.BlockSpec((R, D), lambda i: (i, 0)),
            pl.BlockSpec((R, D), lambda i: (i, 0)),
            pl.BlockSpec((R, NC), lambda i: (i, 0)),
            pl.BlockSpec((1, D), lambda i: (0, 0)),
        ],
        out_specs=pl.BlockSpec((R, D), lambda i: (i, 0)),
        out_shape=jax.ShapeDtypeStruct((N, D), jnp.float32),
    )(p2, y2, degp, b2r)


def kernel(x, edge_index, W1, b1, W2, b2):
    ei = edge_index.astype(jnp.int32)
    src_t = ei[0].reshape(NS, EPT)
    src2 = 2 * src_t
    srco = jnp.stack([src2[:, :NCH2 * KA], src2[:, :NCH2 * KA] + 1], axis=0)
    srct = jnp.stack([src2[:, NCH2 * KA:], src2[:, NCH2 * KA:] + 1], axis=0)
    dst_r = ei[1].reshape(NS, EPT)
    dst_a = dst_r[:, :NCH2 * KA].reshape(NS, NCH2, KA)
    dst_t = dst_r[:, NCH2 * KA:].reshape(NS, 1, KT)
    dst_deg = ei[1].reshape(NC, NS, NCH, K)
    zeros_np = jnp.zeros((NP,), jnp.float32)
    zeros_nd = jnp.zeros((N, D2), jnp.float32)
    ones_k = jnp.ones((K,), jnp.float32)
    iota = jnp.arange(N, dtype=jnp.int32).reshape(NS, NWB, WBK)
    wbidx = jnp.stack([2 * iota, 2 * iota + 1], axis=0)
    srco, srct, wbidx = lax.optimization_barrier((srco, srct, wbidx))

    degp = _sc_deg(dst_deg, zeros_np, ones_k).T
    y1 = _tc_scale_matmul(x, W1, degp)
    p1 = _sc_agg(y1.reshape(2 * N, D2), srco, srct, dst_a, dst_t,
                 zeros_nd, wbidx)
    y2 = _tc_mid(p1.reshape(N, D), y1, degp, b1.reshape(1, D), W2)
    p2 = _sc_agg(y2.reshape(2 * N, D2), srco, srct, dst_a, dst_t,
                 zeros_nd, wbidx)
    return _tc_final(p2.reshape(N, D), y2, degp, b2.reshape(1, D))


# speedup vs baseline: 34.0268x; 1.0052x over previous
import functools

import jax
import jax.numpy as jnp
from jax import lax
from jax.experimental import pallas as pl
from jax.experimental.pallas import tpu as pltpu
from jax.experimental.pallas import tpu_sc as plsc

N = 10000
NP = 10240
E = 320000
D = 128
D2 = 64
NC = 2
NS = 16
K = 80
NCH = (E // (NC * NS)) // K
EPT = E // NS
KA = 128
NCH2 = EPT // KA
KT = EPT - NCH2 * KA
RPT = NP // NS
RPA = N // NS
NB = 4
NWB = 5
WBK = RPA // NWB
NBD = 5

_mesh = plsc.VectorSubcoreMesh(
    core_axis_name="c", subcore_axis_name="s", num_cores=NC, num_subcores=NS
)


def _sc_deg(dst_r, zeros_np, ones_k):
    @functools.partial(
        pl.kernel,
        out_type=jax.ShapeDtypeStruct((NC, NP), jnp.float32),
        mesh=_mesh,
        scratch_types=[
            pltpu.VMEM((NCH, K), jnp.int32),
            pltpu.VMEM((K,), jnp.float32),
            pltpu.VMEM_SHARED((NP,), jnp.float32),
            [pltpu.SemaphoreType.DMA for _ in range(NBD)],
        ],
    )
    def k(dst_hbm, zeros_hbm, ones_hbm, out_hbm, idx_v, ones_v, acc, dsems):
        c = lax.axis_index("c")
        s = lax.axis_index("s")
        base = s * RPT
        inits = [
            pltpu.async_copy(zeros_hbm.at[pl.ds(base, RPT)],
                             acc.at[pl.ds(base, RPT)], dsems[0]),
            pltpu.async_copy(ones_hbm, ones_v, dsems[1]),
            pltpu.async_copy(dst_hbm.at[c, s], idx_v, dsems[2]),
        ]
        for d in inits:
            d.wait()
        plsc.subcore_barrier()

        def group(o, carry):
            jo = o * NBD
            descs = [
                pltpu.async_copy(
                    ones_v, acc.at[idx_v.at[jo + b]], dsems[b], add=True
                )
                for b in range(NBD)
            ]
            for d in descs:
                d.wait()
            return carry

        lax.fori_loop(0, NCH // NBD, group, 0)
        plsc.subcore_barrier()
        pltpu.sync_copy(acc.at[pl.ds(base, RPT)], out_hbm.at[c, pl.ds(base, RPT)])

    return k(dst_r, zeros_np, ones_k)


def _sc_agg(yv, srco, srct, dst_a, dst_t, zeros_nd, wbidx):
    @functools.partial(
        pl.kernel,
        out_type=jax.ShapeDtypeStruct((2 * N, D2), jnp.float32),
        mesh=_mesh,
        scratch_types=[
            pltpu.VMEM((NCH2 * KA,), jnp.int32),
            pltpu.VMEM((KT,), jnp.int32),
            pltpu.VMEM((NCH2, KA), jnp.int32),
            pltpu.VMEM((1, KT), jnp.int32),
            pltpu.VMEM((NWB, WBK), jnp.int32),
            [pltpu.VMEM((WBK, D2), jnp.float32) for _ in range(2)],
            [pltpu.VMEM((KA, D2), jnp.float32) for _ in range(NB)],
            pltpu.VMEM_SHARED((N, D2), jnp.float32),
            [pltpu.SemaphoreType.DMA for _ in range(NB)],
            [pltpu.SemaphoreType.DMA for _ in range(NB)],
        ],
        compiler_params=pltpu.CompilerParams(use_tc_tiling_on_sc=False),
    )
    def k(y_hbm, src_hbm, srct_hbm, dst_hbm, dstt_hbm, zeros_hbm, wb_hbm,
          out_hbm, sidx, sidxt, didx, didxt, wbi, wbufs, gbufs, acc,
          gsems, tsems):
        c = lax.axis_index("c")
        s = lax.axis_index("s")
        base = s * RPA
        inits = [
            pltpu.async_copy(zeros_hbm.at[pl.ds(base, RPA)],
                             acc.at[pl.ds(base, RPA)], gsems[0]),
            pltpu.async_copy(src_hbm.at[c, s], sidx, gsems[1]),
            pltpu.async_copy(srct_hbm.at[c, s], sidxt, gsems[2]),
            pltpu.async_copy(dst_hbm.at[s], didx, gsems[3]),
            pltpu.async_copy(dstt_hbm.at[s], didxt, tsems[0]),
            pltpu.async_copy(wb_hbm.at[c, s], wbi, tsems[1]),
        ]
        for d in inits:
            d.wait()
        plsc.subcore_barrier()

        for b in range(NB):
            pltpu.async_copy(
                y_hbm.at[sidx.at[pl.ds(b * KA, KA)]], gbufs[b], gsems[b]
            )

        def group(jo, last):
            scats = []
            for b in range(NB):
                pltpu.make_async_copy(
                    y_hbm.at[sidx.at[pl.ds((jo + b) * KA, KA)]], gbufs[b],
                    gsems[b]
                ).wait()
                scats.append(pltpu.async_copy(
                    gbufs[b], acc.at[didx.at[jo + b]], tsems[b], add=True
                ))
            for b in range(NB):
                scats[b].wait()
                if not last:
                    pltpu.async_copy(
                        y_hbm.at[sidx.at[pl.ds((jo + NB + b) * KA, KA)]],
                        gbufs[b], gsems[b]
                    )

        def outer(o, carry):
            group(o * NB, last=False)
            return carry

        lax.fori_loop(0, NCH2 // NB - 1, outer, 0)
        group(NCH2 - NB, last=True)
        pltpu.async_copy(
            y_hbm.at[sidxt], gbufs[0].at[pl.ds(0, KT)], gsems[0]
        ).wait()
        pltpu.async_copy(
            gbufs[0].at[pl.ds(0, KT)], acc.at[didxt.at[0]], tsems[0], add=True
        ).wait()
        plsc.subcore_barrier()
        scouts = [None, None]
        for tch in range(NWB):
            w = wbufs[tch % 2]
            if scouts[tch % 2] is not None:
                scouts[tch % 2].wait()
            pltpu.sync_copy(acc.at[pl.ds(base + tch * WBK, WBK)], w)
            scouts[tch % 2] = pltpu.async_copy(
                w, out_hbm.at[wbi.at[tch]], tsems[tch % 2]
            )
        for d in scouts:
            d.wait()

    return k(yv, srco, srct, dst_a, dst_t, zeros_nd, wbidx)


R = 1000
GRID = N // R


def _tc_scale_matmul(x, W1, degp):
    def body(x_ref, w_ref, p_ref, o_ref):
        p = p_ref[...]
        z = lax.rsqrt(p[:, 0] + p[:, 1] + 1.0)
        o_ref[...] = (
            jnp.dot(x_ref[...], w_ref[...], preferred_element_type=jnp.float32)
            * z[:, None]
        )

    return pl.pallas_call(
        body,
        grid=(GRID,),
        in_specs=[
            pl.BlockSpec((R, D), lambda i: (i, 0)),
            pl.BlockSpec((D, D), lambda i: (0, 0)),
            pl.BlockSpec((R, NC), lambda i: (i, 0)),
        ],
        out_specs=pl.BlockSpec((R, D), lambda i: (i, 0)),
        out_shape=jax.ShapeDtypeStruct((N, D), jnp.float32),
    )(x, W1, degp)


def _tc_mid(p1, y1, degp, b1r, W2):
    def body(p_ref, y_ref, dp_ref, b_ref, w_ref, o_ref):
        dp = dp_ref[...]
        z = lax.rsqrt(dp[:, 0] + dp[:, 1] + 1.0)
        agg = p_ref[...] + y_ref[...]
        h = jnp.maximum(agg * z[:, None] + b_ref[...], 0.0)
        o_ref[...] = (
            jnp.dot(h, w_ref[...], preferred_element_type=jnp.float32) * z[:, None]
        )

    return pl.pallas_call(
        body,
        grid=(GRID,),
        in_specs=[
            pl.BlockSpec((R, D), lambda i: (i, 0)),
            pl.BlockSpec((R, D), lambda i: (i, 0)),
            pl.BlockSpec((R, NC), lambda i: (i, 0)),
            pl.BlockSpec((1, D), lambda i: (0, 0)),
            pl.BlockSpec((D, D), lambda i: (0, 0)),
        ],
        out_specs=pl.BlockSpec((R, D), lambda i: (i, 0)),
        out_shape=jax.ShapeDtypeStruct((N, D), jnp.float32),
    )(p1, y1, degp, b1r, W2)


def _tc_final(p2, y2, degp, b2r):
    def body(p_ref, y_ref, dp_ref, b_ref, o_ref):
        dp = dp_ref[...]
        z = lax.rsqrt(dp[:, 0] + dp[:, 1] + 1.0)
        o_ref[...] = (p_ref[...] + y_ref[...]) * z[:, None] + b_ref[...]

    return pl.pallas_call(
        body,
        grid=(GRID,),
        in_specs=[
            pl.BlockSpec((R, D), lambda i: (i, 0)),
            pl.BlockSpec((R, D), lambda i: (i, 0)),
            pl.BlockSpec((R, NC), lambda i: (i, 0)),
            pl.BlockSpec((1, D), lambda i: (0, 0)),
        ],
        out_specs=pl.BlockSpec((R, D), lambda i: (i, 0)),
        out_shape=jax.ShapeDtypeStruct((N, D), jnp.float32),
    )(p2, y2, degp, b2r)


def kernel(x, edge_index, W1, b1, W2, b2):
    ei = edge_index.astype(jnp.int32)
    src_t = ei[0].reshape(NS, EPT)
    src2 = 2 * src_t
    srco = jnp.stack([src2[:, :NCH2 * KA], src2[:, :NCH2 * KA] + 1], axis=0)
    srct = jnp.stack([src2[:, NCH2 * KA:], src2[:, NCH2 * KA:] + 1], axis=0)
    dst_r = ei[1].reshape(NS, EPT)
    dst_a = dst_r[:, :NCH2 * KA].reshape(NS, NCH2, KA)
    dst_t = dst_r[:, NCH2 * KA:].reshape(NS, 1, KT)
    dst_deg = ei[1].reshape(NC, NS, NCH, K)
    zeros_np = jnp.zeros((NP,), jnp.float32)
    zeros_nd = jnp.zeros((N, D2), jnp.float32)
    ones_k = jnp.ones((K,), jnp.float32)
    iota = jnp.arange(N, dtype=jnp.int32).reshape(NS, NWB, WBK)
    wbidx = jnp.stack([2 * iota, 2 * iota + 1], axis=0)
    srco, srct, wbidx = lax.optimization_barrier((srco, srct, wbidx))

    degp = _sc_deg(dst_deg, zeros_np, ones_k).T
    y1 = _tc_scale_matmul(x, W1, degp)
    p1 = _sc_agg(y1.reshape(2 * N, D2), srco, srct, dst_a, dst_t,
                 zeros_nd, wbidx)
    y2 = _tc_mid(p1.reshape(N, D), y1, degp, b1.reshape(1, D), W2)
    p2 = _sc_agg(y2.reshape(2 * N, D2), srco, srct, dst_a, dst_t,
                 zeros_nd, wbidx)
    return _tc_final(p2.reshape(N, D), y2, degp, b2.reshape(1, D))
